# baseline (speedup 1.0000x reference)
import functools

import jax
import jax.numpy as jnp
import numpy as np
from jax import lax
from jax.experimental import pallas as pl
from jax.experimental.pallas import tpu as pltpu
from jax.experimental.pallas import tpu_sc as plsc

_N = 10000
_D = 128
_H = 16
_C = 40
_E = 320000
_W1 = 32

_NC = 2
_NS = 16
_NW = _NC * _NS
_M = 128
_R = 80
_EP = _NW * _R * _M
_NPAD = 10112
_STRIPE = _NPAD // _NS
_K = 8


def _tc_pre_body(x_ref, ws_ref, wn_ref, b1_ref, c1_ref, xs_ref, xn_ref):
    x = x_ref[...]
    xs_ref[...] = jnp.dot(x, ws_ref[...], preferred_element_type=jnp.float32) + b1_ref[...]
    xn_ref[...] = jnp.dot(x, wn_ref[...], preferred_element_type=jnp.float32) + c1_ref[...]


def _tc_pre(x, wsT, wnT, b1r, c1r):
    return pl.pallas_call(
        _tc_pre_body,
        out_shape=[
            jax.ShapeDtypeStruct((_N, _H), jnp.float32),
            jax.ShapeDtypeStruct((_N, _W1), jnp.float32),
        ],
    )(x, wsT, wnT, b1r, c1r)


def _sc_agg(table, src_rows, dst_rows, zeros, width):
    @functools.partial(
        pl.kernel,
        out_type=jax.ShapeDtypeStruct((_NC, _NPAD, width), jnp.float32),
        mesh=plsc.VectorSubcoreMesh(core_axis_name="c", subcore_axis_name="s"),
        compiler_params=pltpu.CompilerParams(use_tc_tiling_on_sc=False),
        scratch_types=[
            pltpu.VMEM((_R, _M), jnp.int32),
            pltpu.VMEM((_R, _M), jnp.int32),
            pltpu.VMEM((_K, _M, width), jnp.float32),
            pltpu.VMEM((_STRIPE, width), jnp.float32),
            pltpu.VMEM_SHARED((_NPAD, width), jnp.float32),
            [pltpu.SemaphoreType.DMA] * _K,
            [pltpu.SemaphoreType.DMA] * _K,
        ],
    )
    def agg(table_hbm, src_hbm, dst_hbm, z_hbm, out_hbm,
            src_v, dst_v, rows_v, strip_v, acc_sh, gsems, ssems):
        c = lax.axis_index("c")
        s = lax.axis_index("s")
        pltpu.sync_copy(z_hbm.at[pl.ds(s * _STRIPE, _STRIPE)], strip_v)
        pltpu.sync_copy(strip_v, acc_sh.at[pl.ds(s * _STRIPE, _STRIPE)])
        plsc.subcore_barrier()
        wid = s * _NC + c
        pltpu.sync_copy(src_hbm.at[pl.ds(wid * _R, _R)], src_v)
        pltpu.sync_copy(dst_hbm.at[pl.ds(wid * _R, _R)], dst_v)

        def pipe(g, carry):
            gd = []
            for b in range(_K):
                j = g * _K + b
                gd.append(pltpu.async_copy(
                    table_hbm.at[src_v.at[j]], rows_v.at[b], gsems[b]))
            sd = []
            for b in range(_K):
                j = g * _K + b
                gd[b].wait()
                sd.append(pltpu.async_copy(
                    rows_v.at[b], acc_sh.at[dst_v.at[j]], ssems[b], add=True))
            for b in range(_K):
                sd[b].wait()
            return carry

        lax.fori_loop(0, _R // _K, pipe, 0)
        plsc.subcore_barrier()
        pltpu.sync_copy(acc_sh.at[pl.ds(s * _STRIPE, _STRIPE)], strip_v)
        pltpu.sync_copy(strip_v, out_hbm.at[c, pl.ds(s * _STRIPE, _STRIPE)])

    return agg(table, src_rows, dst_rows, zeros)


def _tc_mid_body(xs_ref, p_ref, w2s_ref, b2_ref, h_ref, hs_ref):
    p = p_ref[0, :_N] + p_ref[1, :_N]
    den = jnp.maximum(p[:, _H:_H + 1], 1.0)
    h = jnp.maximum(xs_ref[...] + p[:, :_H] / den, 0.0)
    h_ref[...] = h
    hs_ref[...] = jnp.dot(h, w2s_ref[...], preferred_element_type=jnp.float32) + b2_ref[...]


def _tc_mid(xs, p32, w2sT, b2r):
    return pl.pallas_call(
        _tc_mid_body,
        out_shape=[
            jax.ShapeDtypeStruct((_N, _H), jnp.float32),
            jax.ShapeDtypeStruct((_N, _C), jnp.float32),
        ],
    )(xs, p32, w2sT, b2r)


def _tc_out_body(hs_ref, q_ref, p_ref, w2n_ref, o_ref):
    p = p_ref[0, :_N] + p_ref[1, :_N]
    den = jnp.maximum(p[:, _H:_H + 1], 1.0)
    agg = q_ref[0, :_N] + q_ref[1, :_N]
    o_ref[...] = hs_ref[...] + jnp.dot(agg, w2n_ref[...], preferred_element_type=jnp.float32) / den


def _tc_out(hs, q16, p32, w2nT):
    return pl.pallas_call(
        _tc_out_body,
        out_shape=jax.ShapeDtypeStruct((_N, _C), jnp.float32),
    )(hs, q16, p32, w2nT)


_PAD_AR = np.arange(_EP - _E, dtype=np.int32)
_PAD_SRC = ((_PAD_AR * 127) % _N).reshape(-1, _M)
_PAD_DST = (_N + _PAD_AR % (_NPAD - _N)).reshape(-1, _M)


def kernel(x, edge_index, W1_self, W1_neigh, b1, W2_self, W2_neigh, b2):
    ei = edge_index.astype(jnp.int32)
    src_rows = jnp.concatenate([ei[0].reshape(_E // _M, _M), jnp.asarray(_PAD_SRC)])
    dst_rows = jnp.concatenate([ei[1].reshape(_E // _M, _M), jnp.asarray(_PAD_DST)])
    wsT = W1_self.T
    wnT = jnp.zeros((_D, _W1), jnp.float32).at[:, :_H].set(W1_neigh.T)
    c1 = jnp.zeros((1, _W1), jnp.float32).at[0, _H].set(1.0)
    z32 = jnp.zeros((_NPAD, _W1), jnp.float32)
    z16 = jnp.zeros((_NPAD, _H), jnp.float32)

    xs, xn = _tc_pre(x, wsT, wnT, b1.reshape(1, _H), c1)
    p32 = _sc_agg(xn, src_rows, dst_rows, z32, _W1)
    h, hs = _tc_mid(xs, p32, W2_self.T, b2.reshape(1, _C))
    q16 = _sc_agg(h, src_rows, dst_rows, z16, _H)
    return _tc_out(hs, q16, p32, W2_neigh.T)


# speedup vs baseline: 1.0161x; 1.0161x over previous
import functools

import jax
import jax.numpy as jnp
import numpy as np
from jax import lax
from jax.experimental import pallas as pl
from jax.experimental.pallas import tpu as pltpu
from jax.experimental.pallas import tpu_sc as plsc

_N = 10000
_D = 128
_H = 16
_C = 40
_E = 320000
_W1 = 32

_NC = 2
_NS = 16
_NW = _NC * _NS
_M = 128
_R = 80
_EP = _NW * _R * _M
_NPAD = 10112
_STRIPE = _NPAD // _NS
_K = 8


def _tc_pre_body(x_ref, ws_ref, wn4_ref, b1_ref, c14_ref, xs_ref, xnf_ref):
    x = x_ref[...]
    xs_ref[...] = jnp.dot(x, ws_ref[...], preferred_element_type=jnp.float32) + b1_ref[...]
    y = jnp.dot(x, wn4_ref[...], preferred_element_type=jnp.float32) + c14_ref[...]
    grp = lax.broadcasted_iota(jnp.int32, (4, _M), 0)
    lane = lax.broadcasted_iota(jnp.int32, (4, _M), 1) // _W1
    mask = (grp == lane).astype(jnp.float32)
    y3 = y.reshape(_N // 4, 4, _M) * mask[None]
    xnf_ref[...] = y3[:, 0, :] + y3[:, 1, :] + y3[:, 2, :] + y3[:, 3, :]


def _tc_pre(x, wsT, wnT, b1r, c1r):
    return pl.pallas_call(
        _tc_pre_body,
        out_shape=[
            jax.ShapeDtypeStruct((_N, _H), jnp.float32),
            jax.ShapeDtypeStruct((_N * _W1 // _M, _M), jnp.float32),
        ],
    )(x, wsT, jnp.tile(wnT, (1, 4)), b1r, jnp.tile(c1r, (1, 4)))


def _sc_agg(table, src_rows, dst_rows, zeros, width):
    @functools.partial(
        pl.kernel,
        out_type=jax.ShapeDtypeStruct((_NC, _NPAD, width), jnp.float32),
        mesh=plsc.VectorSubcoreMesh(core_axis_name="c", subcore_axis_name="s"),
        compiler_params=pltpu.CompilerParams(use_tc_tiling_on_sc=False),
        scratch_types=[
            pltpu.VMEM((_R, _M), jnp.int32),
            pltpu.VMEM((_R, _M), jnp.int32),
            pltpu.VMEM((_K, _M, width), jnp.float32),
            pltpu.VMEM((_STRIPE, width), jnp.float32),
            pltpu.VMEM_SHARED((_NPAD, width), jnp.float32),
            [pltpu.SemaphoreType.DMA] * _K,
            [pltpu.SemaphoreType.DMA] * _K,
        ],
    )
    def agg(table_hbm, src_hbm, dst_hbm, z_hbm, out_hbm,
            src_v, dst_v, rows_v, strip_v, acc_sh, gsems, ssems):
        c = lax.axis_index("c")
        s = lax.axis_index("s")
        pltpu.sync_copy(z_hbm.at[pl.ds(s * _STRIPE, _STRIPE)], strip_v)
        pltpu.sync_copy(strip_v, acc_sh.at[pl.ds(s * _STRIPE, _STRIPE)])
        plsc.subcore_barrier()
        wid = s * _NC + c
        pltpu.sync_copy(src_hbm.at[pl.ds(wid * _R, _R)], src_v)
        pltpu.sync_copy(dst_hbm.at[pl.ds(wid * _R, _R)], dst_v)

        def pipe(g, carry):
            gd = []
            for b in range(_K):
                j = g * _K + b
                gd.append(pltpu.async_copy(
                    table_hbm.at[src_v.at[j]], rows_v.at[b], gsems[b]))
            sd = []
            for b in range(_K):
                j = g * _K + b
                gd[b].wait()
                sd.append(pltpu.async_copy(
                    rows_v.at[b], acc_sh.at[dst_v.at[j]], ssems[b], add=True))
            for b in range(_K):
                sd[b].wait()
            return carry

        lax.fori_loop(0, _R // _K, pipe, 0)
        plsc.subcore_barrier()
        pltpu.sync_copy(acc_sh.at[pl.ds(s * _STRIPE, _STRIPE)], strip_v)
        pltpu.sync_copy(strip_v, out_hbm.at[c, pl.ds(s * _STRIPE, _STRIPE)])

    return agg(table, src_rows, dst_rows, zeros)


def _tc_mid_body(xs_ref, p_ref, w2s_ref, b2_ref, h_ref, hs_ref):
    p = p_ref[0, :_N] + p_ref[1, :_N]
    den = jnp.maximum(p[:, _H:_H + 1], 1.0)
    h = jnp.maximum(xs_ref[...] + p[:, :_H] / den, 0.0)
    h_ref[...] = h
    hs_ref[...] = jnp.dot(h, w2s_ref[...], preferred_element_type=jnp.float32) + b2_ref[...]


def _tc_mid(xs, p32, w2sT, b2r):
    return pl.pallas_call(
        _tc_mid_body,
        out_shape=[
            jax.ShapeDtypeStruct((_N, _H), jnp.float32),
            jax.ShapeDtypeStruct((_N, _C), jnp.float32),
        ],
    )(xs, p32, w2sT, b2r)


def _tc_out_body(hs_ref, q_ref, p_ref, w2n_ref, o_ref):
    p = p_ref[0, :_N] + p_ref[1, :_N]
    den = jnp.maximum(p[:, _H:_H + 1], 1.0)
    agg = q_ref[0, :_N] + q_ref[1, :_N]
    o_ref[...] = hs_ref[...] + jnp.dot(agg, w2n_ref[...], preferred_element_type=jnp.float32) / den


def _tc_out(hs, q16, p32, w2nT):
    return pl.pallas_call(
        _tc_out_body,
        out_shape=jax.ShapeDtypeStruct((_N, _C), jnp.float32),
    )(hs, q16, p32, w2nT)


_PAD_AR = np.arange(_EP - _E, dtype=np.int32)
_PAD_SRC = ((_PAD_AR * 127) % _N).reshape(-1, _M)
_PAD_DST = (_N + _PAD_AR % (_NPAD - _N)).reshape(-1, _M)


def kernel(x, edge_index, W1_self, W1_neigh, b1, W2_self, W2_neigh, b2):
    ei = edge_index.astype(jnp.int32)
    src_rows = jnp.concatenate([ei[0].reshape(_E // _M, _M), jnp.asarray(_PAD_SRC)])
    dst_rows = jnp.concatenate([ei[1].reshape(_E // _M, _M), jnp.asarray(_PAD_DST)])
    wsT = W1_self.T
    wnT = jnp.zeros((_D, _W1), jnp.float32).at[:, :_H].set(W1_neigh.T)
    c1 = jnp.zeros((1, _W1), jnp.float32).at[0, _H].set(1.0)
    z32 = jnp.zeros((_NPAD, _W1), jnp.float32)
    z16 = jnp.zeros((_NPAD, _H), jnp.float32)

    xs, xnf = _tc_pre(x, wsT, wnT, b1.reshape(1, _H), c1)
    p32 = _sc_agg(xnf.reshape(_N, _W1), src_rows, dst_rows, z32, _W1)
    h, hs = _tc_mid(xs, p32, W2_self.T, b2.reshape(1, _C))
    q16 = _sc_agg(h, src_rows, dst_rows, z16, _H)
    return _tc_out(hs, q16, p32, W2_neigh.T)


# speedup vs baseline: 1.2494x; 1.2296x over previous
import functools

import jax
import jax.numpy as jnp
import numpy as np
from jax import lax
from jax.experimental import pallas as pl
from jax.experimental.pallas import tpu as pltpu
from jax.experimental.pallas import tpu_sc as plsc

_N = 10000
_D = 128
_H = 16
_C = 40
_E = 320000
_W1 = 32

_NC = 2
_NS = 16
_NW = _NC * _NS
_M = 128
_R = 80
_EP = _NW * _R * _M
_NPAD = 10112
_STRIPE = _NPAD // _NS
_K = 8


def _tc_pre_body(x_ref, ws_ref, wn4_ref, b1_ref, c14_ref, xs_ref, xnf_ref):
    x = x_ref[...]
    xs_ref[...] = jnp.dot(x, ws_ref[...], preferred_element_type=jnp.float32) + b1_ref[...]
    y = jnp.dot(x, wn4_ref[...], preferred_element_type=jnp.float32) + c14_ref[...]
    grp = lax.broadcasted_iota(jnp.int32, (4, _M), 0)
    lane = lax.broadcasted_iota(jnp.int32, (4, _M), 1) // _W1
    mask = (grp == lane).astype(jnp.float32)
    y3 = y.reshape(_N // 4, 4, _M) * mask[None]
    xnf_ref[...] = y3[:, 0, :] + y3[:, 1, :] + y3[:, 2, :] + y3[:, 3, :]


def _tc_pre(x, wsT, wnT, b1r, c1r):
    return pl.pallas_call(
        _tc_pre_body,
        out_shape=[
            jax.ShapeDtypeStruct((_N, _H), jnp.float32),
            jax.ShapeDtypeStruct((_N * _W1 // _M, _M), jnp.float32),
        ],
    )(x, wsT, jnp.tile(wnT, (1, 4)), b1r, jnp.tile(c1r, (1, 4)))


def _sc_agg(table, src_rows, dst_rows, zeros, width):
    @functools.partial(
        pl.kernel,
        out_type=jax.ShapeDtypeStruct((_NC, _NPAD, width), jnp.float32),
        mesh=plsc.VectorSubcoreMesh(core_axis_name="c", subcore_axis_name="s"),
        compiler_params=pltpu.CompilerParams(use_tc_tiling_on_sc=False),
        scratch_types=[
            pltpu.VMEM((_R, _M), jnp.int32),
            pltpu.VMEM((_R, _M), jnp.int32),
            pltpu.VMEM((_K, _M, width), jnp.float32),
            pltpu.VMEM((_STRIPE, width), jnp.float32),
            pltpu.VMEM_SHARED((_NPAD, width), jnp.float32),
            [pltpu.SemaphoreType.DMA] * _K,
            [pltpu.SemaphoreType.DMA] * _K,
        ],
    )
    def agg(table_hbm, src_hbm, dst_hbm, z_hbm, out_hbm,
            src_v, dst_v, rows_v, strip_v, acc_sh, gsems, ssems):
        c = lax.axis_index("c")
        s = lax.axis_index("s")
        pltpu.sync_copy(z_hbm.at[pl.ds(s * _STRIPE, _STRIPE)], strip_v)
        pltpu.sync_copy(strip_v, acc_sh.at[pl.ds(s * _STRIPE, _STRIPE)])
        plsc.subcore_barrier()
        wid = s * _NC + c
        pltpu.sync_copy(src_hbm.at[pl.ds(wid * _R, _R)], src_v)
        pltpu.sync_copy(dst_hbm.at[pl.ds(wid * _R, _R)], dst_v)

        def pipe(g, carry):
            gd = []
            for b in range(_K):
                j = g * _K + b
                gd.append(pltpu.async_copy(
                    table_hbm.at[src_v.at[j]], rows_v.at[b], gsems[b]))
            sd = []
            for b in range(_K):
                j = g * _K + b
                gd[b].wait()
                sd.append(pltpu.async_copy(
                    rows_v.at[b], acc_sh.at[dst_v.at[j]], ssems[b], add=True))
            for b in range(_K):
                sd[b].wait()
            return carry

        lax.fori_loop(0, _R // _K, pipe, 0)
        plsc.subcore_barrier()
        pltpu.sync_copy(acc_sh.at[pl.ds(s * _STRIPE, _STRIPE)], strip_v)
        pltpu.sync_copy(strip_v, out_hbm.at[c, pl.ds(s * _STRIPE, _STRIPE)])

    return agg(table, src_rows, dst_rows, zeros)


def _unpack_ym(pf, width):
    g = _M // width
    half = _NPAD * width // _M
    sf = pf[:half] + pf[half:]
    y = jnp.broadcast_to(sf[:, None, :], (half, g, _M)).reshape(_NPAD, _M)
    rowg = lax.broadcasted_iota(jnp.int32, (_NPAD, _M), 0) % g
    laneg = lax.broadcasted_iota(jnp.int32, (_NPAD, _M), 1) // width
    return jnp.where(rowg == laneg, y, 0.0)


def _coll(width):
    return (lax.broadcasted_iota(jnp.int32, (_M, width), 0) % width
            == lax.broadcasted_iota(jnp.int32, (_M, width), 1)).astype(jnp.float32)


def _tc_mid_body(xs_ref, p_ref, w2s_ref, b2t_ref, r8_ref, hf_ref, hst_ref):
    p = jnp.dot(_unpack_ym(p_ref[...], _W1), _coll(_W1),
                preferred_element_type=jnp.float32)[:_N]
    den = jnp.maximum(p[:, _H:_H + 1], 1.0)
    h = jnp.maximum(xs_ref[...] + p[:, :_H] / den, 0.0)
    y = jnp.dot(h, r8_ref[...], preferred_element_type=jnp.float32)
    grp = lax.broadcasted_iota(jnp.int32, (8, _M), 0)
    lane = lax.broadcasted_iota(jnp.int32, (8, _M), 1) // _H
    mask = (grp == lane).astype(jnp.float32)
    y3 = y.reshape(_N // 8, 8, _M) * mask[None]
    hf = y3[:, 0, :]
    for k in range(1, 8):
        hf = hf + y3[:, k, :]
    hf_ref[...] = hf
    hst_ref[...] = lax.dot_general(
        w2s_ref[...], h, (((1,), (1,)), ((), ())),
        preferred_element_type=jnp.float32) + b2t_ref[...]


def _tc_mid(xs, p32f, w2s, b2t, r8):
    return pl.pallas_call(
        _tc_mid_body,
        out_shape=[
            jax.ShapeDtypeStruct((_N * _H // _M, _M), jnp.float32),
            jax.ShapeDtypeStruct((_C, _N), jnp.float32),
        ],
    )(xs, p32f, w2s, b2t, r8)


def _tc_out_body(hst_ref, q_ref, p_ref, w2n_ref, sel_ref, o_ref):
    ymp = _unpack_ym(p_ref[...], _W1)
    cntt = lax.dot_general(sel_ref[...], ymp, (((1,), (1,)), ((), ())),
                           preferred_element_type=jnp.float32)[:, :_N]
    dent = jnp.maximum(cntt, 1.0)
    ymq = _unpack_ym(q_ref[...], _H)
    aggt = lax.dot_general(_coll(_H), ymq, (((0,), (1,)), ((), ())),
                           preferred_element_type=jnp.float32)[:, :_N]
    o_ref[...] = hst_ref[...] + jnp.dot(
        w2n_ref[...], aggt, preferred_element_type=jnp.float32) / dent


def _tc_out(hst, q16f, p32f, w2n, sel):
    return pl.pallas_call(
        _tc_out_body,
        out_shape=jax.ShapeDtypeStruct((_C, _N), jnp.float32),
    )(hst, q16f, p32f, w2n, sel)


_PAD_AR = np.arange(_EP - _E, dtype=np.int32)
_PAD_SRC = ((_PAD_AR * 127) % _N).reshape(-1, _M)
_PAD_DST = (_N + _PAD_AR % (_NPAD - _N)).reshape(-1, _M)


def kernel(x, edge_index, W1_self, W1_neigh, b1, W2_self, W2_neigh, b2):
    ei = edge_index.astype(jnp.int32)
    src_rows = jnp.concatenate([ei[0].reshape(_E // _M, _M), jnp.asarray(_PAD_SRC)])
    dst_rows = jnp.concatenate([ei[1].reshape(_E // _M, _M), jnp.asarray(_PAD_DST)])
    wsT = W1_self.T
    wnT = jnp.zeros((_D, _W1), jnp.float32).at[:, :_H].set(W1_neigh.T)
    c1 = jnp.zeros((1, _W1), jnp.float32).at[0, _H].set(1.0)
    z32 = jnp.zeros((_NPAD, _W1), jnp.float32)
    z16 = jnp.zeros((_NPAD, _H), jnp.float32)

    r8 = jnp.tile(jnp.eye(_H, dtype=jnp.float32), (1, _M // _H))
    sel = jnp.zeros((1, _M), jnp.float32).at[0, _H::_W1].set(1.0)

    xs, xnf = _tc_pre(x, wsT, wnT, b1.reshape(1, _H), c1)
    p32 = _sc_agg(xnf.reshape(_N, _W1), src_rows, dst_rows, z32, _W1)
    p32f = p32.reshape(_NC * _NPAD * _W1 // _M, _M)
    hf, hst = _tc_mid(xs, p32f, W2_self, b2.reshape(_C, 1), r8)
    q16 = _sc_agg(hf.reshape(_N, _H), src_rows, dst_rows, z16, _H)
    q16f = q16.reshape(_NC * _NPAD * _H // _M, _M)
    outt = _tc_out(hst, q16f, p32f, W2_neigh, sel)
    return outt.T


# speedup vs baseline: 1.3400x; 1.0725x over previous
import functools

import jax
import jax.numpy as jnp
import numpy as np
from jax import lax
from jax.experimental import pallas as pl
from jax.experimental.pallas import tpu as pltpu
from jax.experimental.pallas import tpu_sc as plsc

_N = 10000
_D = 128
_H = 16
_C = 40
_E = 320000
_W1 = 32

_NC = 2
_NS = 16
_NW = _NC * _NS
_M = 128
_R = 80
_EP = _NW * _R * _M
_NPAD = 10112
_STRIPE = _NPAD // _NS
_K = 8


def _tc_pre_body(x_ref, ws_ref, wn4_ref, b1_ref, c14_ref, xs_ref, xnf_ref):
    x = x_ref[...]
    xs_ref[...] = jnp.dot(x, ws_ref[...], preferred_element_type=jnp.float32) + b1_ref[...]
    y = jnp.dot(x, wn4_ref[...], preferred_element_type=jnp.float32) + c14_ref[...]
    grp = lax.broadcasted_iota(jnp.int32, (4, _M), 0)
    lane = lax.broadcasted_iota(jnp.int32, (4, _M), 1) // _W1
    mask = (grp == lane).astype(jnp.float32)
    y3 = y.reshape(_N // 4, 4, _M) * mask[None]
    xnf_ref[...] = y3[:, 0, :] + y3[:, 1, :] + y3[:, 2, :] + y3[:, 3, :]


def _tc_pre(x, wsT, wnT, b1r, c1r):
    return pl.pallas_call(
        _tc_pre_body,
        out_shape=[
            jax.ShapeDtypeStruct((_N, _H), jnp.float32),
            jax.ShapeDtypeStruct((_N * _W1 // _M, _M), jnp.float32),
        ],
    )(x, wsT, jnp.tile(wnT, (1, 4)), b1r, jnp.tile(c1r, (1, 4)))


def _sc_agg(table, src_rows, dst_rows, zeros, width):
    @functools.partial(
        pl.kernel,
        out_type=jax.ShapeDtypeStruct((_NC, _NPAD, width), jnp.float32),
        mesh=plsc.VectorSubcoreMesh(core_axis_name="c", subcore_axis_name="s"),
        compiler_params=pltpu.CompilerParams(use_tc_tiling_on_sc=False),
        scratch_types=[
            pltpu.VMEM((_R, _M), jnp.int32),
            pltpu.VMEM((_R, _M), jnp.int32),
            pltpu.VMEM((_K, _M, width), jnp.float32),
            pltpu.VMEM((_STRIPE, width), jnp.float32),
            pltpu.VMEM_SHARED((_NPAD, width), jnp.float32),
            [pltpu.SemaphoreType.DMA] * _K,
            [pltpu.SemaphoreType.DMA] * _K,
        ],
    )
    def agg(table_hbm, src_hbm, dst_hbm, z_hbm, out_hbm,
            src_v, dst_v, rows_v, strip_v, acc_sh, gsems, ssems):
        c = lax.axis_index("c")
        s = lax.axis_index("s")
        pltpu.sync_copy(z_hbm.at[pl.ds(s * _STRIPE, _STRIPE)], strip_v)
        pltpu.sync_copy(strip_v, acc_sh.at[pl.ds(s * _STRIPE, _STRIPE)])
        plsc.subcore_barrier()
        wid = s * _NC + c
        pltpu.sync_copy(src_hbm.at[pl.ds(wid * _R, _R)], src_v)
        pltpu.sync_copy(dst_hbm.at[pl.ds(wid * _R, _R)], dst_v)

        for b in range(_K):
            pltpu.async_copy(table_hbm.at[src_v.at[b]], rows_v.at[b], gsems[b])

        def pipe(g, carry):
            base = g * _K
            for b in range(_K):
                j = base + b
                pltpu.make_async_copy(
                    table_hbm.at[src_v.at[j]], rows_v.at[b], gsems[b]).wait()
                pltpu.async_copy(
                    rows_v.at[b], acc_sh.at[dst_v.at[j]], ssems[b], add=True)
            for b in range(_K):
                j = base + b
                pltpu.make_async_copy(
                    rows_v.at[b], acc_sh.at[dst_v.at[j]], ssems[b]).wait()
                pltpu.async_copy(
                    table_hbm.at[src_v.at[j + _K]], rows_v.at[b], gsems[b])
            return carry

        lax.fori_loop(0, _R // _K - 1, pipe, 0)
        base = _R - _K
        for b in range(_K):
            j = base + b
            pltpu.make_async_copy(
                table_hbm.at[src_v.at[j]], rows_v.at[b], gsems[b]).wait()
            pltpu.async_copy(
                rows_v.at[b], acc_sh.at[dst_v.at[j]], ssems[b], add=True)
        for b in range(_K):
            j = base + b
            pltpu.make_async_copy(
                rows_v.at[b], acc_sh.at[dst_v.at[j]], ssems[b]).wait()
        plsc.subcore_barrier()
        pltpu.sync_copy(acc_sh.at[pl.ds(s * _STRIPE, _STRIPE)], strip_v)
        pltpu.sync_copy(strip_v, out_hbm.at[c, pl.ds(s * _STRIPE, _STRIPE)])

    return agg(table, src_rows, dst_rows, zeros)


def _unpack_ym(pf, width):
    g = _M // width
    half = _NPAD * width // _M
    sf = pf[:half] + pf[half:]
    y = jnp.broadcast_to(sf[:, None, :], (half, g, _M)).reshape(_NPAD, _M)
    rowg = lax.broadcasted_iota(jnp.int32, (_NPAD, _M), 0) % g
    laneg = lax.broadcasted_iota(jnp.int32, (_NPAD, _M), 1) // width
    return jnp.where(rowg == laneg, y, 0.0)


def _coll(width):
    return (lax.broadcasted_iota(jnp.int32, (_M, width), 0) % width
            == lax.broadcasted_iota(jnp.int32, (_M, width), 1)).astype(jnp.float32)


def _tc_mid_body(xs_ref, p_ref, w2s_ref, b2t_ref, r8_ref, hf_ref, hst_ref):
    p = jnp.dot(_unpack_ym(p_ref[...], _W1), _coll(_W1),
                preferred_element_type=jnp.float32)[:_N]
    den = jnp.maximum(p[:, _H:_H + 1], 1.0)
    h = jnp.maximum(xs_ref[...] + p[:, :_H] / den, 0.0)
    y = jnp.dot(h, r8_ref[...], preferred_element_type=jnp.float32)
    grp = lax.broadcasted_iota(jnp.int32, (8, _M), 0)
    lane = lax.broadcasted_iota(jnp.int32, (8, _M), 1) // _H
    mask = (grp == lane).astype(jnp.float32)
    y3 = y.reshape(_N // 8, 8, _M) * mask[None]
    hf = y3[:, 0, :]
    for k in range(1, 8):
        hf = hf + y3[:, k, :]
    hf_ref[...] = hf
    hst_ref[...] = lax.dot_general(
        w2s_ref[...], h, (((1,), (1,)), ((), ())),
        preferred_element_type=jnp.float32) + b2t_ref[...]


def _tc_mid(xs, p32f, w2s, b2t, r8):
    return pl.pallas_call(
        _tc_mid_body,
        out_shape=[
            jax.ShapeDtypeStruct((_N * _H // _M, _M), jnp.float32),
            jax.ShapeDtypeStruct((_C, _N), jnp.float32),
        ],
    )(xs, p32f, w2s, b2t, r8)


def _tc_out_body(hst_ref, q_ref, p_ref, w2n_ref, sel_ref, o_ref):
    ymp = _unpack_ym(p_ref[...], _W1)
    cntt = lax.dot_general(sel_ref[...], ymp, (((1,), (1,)), ((), ())),
                           preferred_element_type=jnp.float32)[:, :_N]
    dent = jnp.maximum(cntt, 1.0)
    ymq = _unpack_ym(q_ref[...], _H)
    aggt = lax.dot_general(_coll(_H), ymq, (((0,), (1,)), ((), ())),
                           preferred_element_type=jnp.float32)[:, :_N]
    o_ref[...] = hst_ref[...] + jnp.dot(
        w2n_ref[...], aggt, preferred_element_type=jnp.float32) / dent


def _tc_out(hst, q16f, p32f, w2n, sel):
    return pl.pallas_call(
        _tc_out_body,
        out_shape=jax.ShapeDtypeStruct((_C, _N), jnp.float32),
    )(hst, q16f, p32f, w2n, sel)


_PAD_AR = np.arange(_EP - _E, dtype=np.int32)
_PAD_SRC = ((_PAD_AR * 127) % _N).reshape(-1, _M)
_PAD_DST = (_N + _PAD_AR % (_NPAD - _N)).reshape(-1, _M)


def kernel(x, edge_index, W1_self, W1_neigh, b1, W2_self, W2_neigh, b2):
    ei = edge_index.astype(jnp.int32)
    src_rows = jnp.concatenate([ei[0].reshape(_E // _M, _M), jnp.asarray(_PAD_SRC)])
    dst_rows = jnp.concatenate([ei[1].reshape(_E // _M, _M), jnp.asarray(_PAD_DST)])
    wsT = W1_self.T
    wnT = jnp.zeros((_D, _W1), jnp.float32).at[:, :_H].set(W1_neigh.T)
    c1 = jnp.zeros((1, _W1), jnp.float32).at[0, _H].set(1.0)
    z32 = jnp.zeros((_NPAD, _W1), jnp.float32)
    z16 = jnp.zeros((_NPAD, _H), jnp.float32)

    r8 = jnp.tile(jnp.eye(_H, dtype=jnp.float32), (1, _M // _H))
    sel = jnp.zeros((1, _M), jnp.float32).at[0, _H::_W1].set(1.0)

    xs, xnf = _tc_pre(x, wsT, wnT, b1.reshape(1, _H), c1)
    p32 = _sc_agg(xnf.reshape(_N, _W1), src_rows, dst_rows, z32, _W1)
    p32f = p32.reshape(_NC * _NPAD * _W1 // _M, _M)
    hf, hst = _tc_mid(xs, p32f, W2_self, b2.reshape(_C, 1), r8)
    q16 = _sc_agg(hf.reshape(_N, _H), src_rows, dst_rows, z16, _H)
    q16f = q16.reshape(_NC * _NPAD * _H // _M, _M)
    outt = _tc_out(hst, q16f, p32f, W2_neigh, sel)
    return outt.T
